# baseline (speedup 1.0000x reference)
import functools
import jax
import jax.numpy as jnp
from jax import lax
from jax.experimental import pallas as pl
from jax.experimental.pallas import tpu as pltpu
from jax.experimental.pallas import tpu_sc as plsc

DIM = 64
SEQ = 200
BATCH = 1024
NROWS = BATCH * SEQ
NW = 32
CHUNK = 128
NCHUNKS = NROWS // CHUNK
CH_PW = NCHUNKS // NW
SCALE = 8.0

_mesh = plsc.VectorSubcoreMesh(core_axis_name="c", subcore_axis_name="s")


@functools.partial(
    pl.kernel,
    mesh=_mesh,
    out_type=jax.ShapeDtypeStruct((NROWS, DIM), jnp.float32),
    scratch_types=[
        pltpu.VMEM((CH_PW, CHUNK), jnp.int32),
        pltpu.VMEM((CH_PW, CHUNK), jnp.int32),
        pltpu.VMEM((2 * SEQ, DIM), jnp.float32),
        pltpu.VMEM((CHUNK, 2 * DIM), jnp.float32),
        pltpu.VMEM((CHUNK, 2 * DIM), jnp.float32),
        pltpu.VMEM((CHUNK, DIM), jnp.float32),
        pltpu.SemaphoreType.DMA,
        pltpu.SemaphoreType.DMA,
    ],
)
def _emb_kernel(idx_hbm, off_hbm, pe_hbm, tab_hbm, out_hbm, idx_v, off_v,
                pe_v, buf0, buf1, obuf, s0, s1):
    wid = lax.axis_index("s") * 2 + lax.axis_index("c")

    pltpu.sync_copy(idx_hbm.at[wid], idx_v)
    pltpu.sync_copy(off_hbm.at[wid], off_v)
    pltpu.sync_copy(pe_hbm, pe_v)

    pltpu.async_copy(tab_hbm.at[idx_v.at[0]], buf0, s0)
    pltpu.async_copy(tab_hbm.at[idx_v.at[1]], buf1, s1)

    def pair(c2, carry):
        for k, (buf, sem) in enumerate(((buf0, s0), (buf1, s1))):
            c = c2 * 2 + k
            pltpu.make_async_copy(tab_hbm.at[idx_v.at[0]], buf, sem).wait()
            r0 = (wid * CH_PW + c) * CHUNK
            p0 = lax.rem(r0, SEQ)

            def body(g, _):
                off16 = off_v[c, pl.ds(g * 16, 16)]
                for i in range(16):
                    off = off16[i]
                    r = g * 16 + i
                    pr = p0 + r
                    for j in range(DIM // 16):
                        obuf[r, pl.ds(j * 16, 16)] = (
                            buf[r, pl.ds(off + j * 16, 16)] * SCALE
                            + pe_v[pr, pl.ds(j * 16, 16)]
                        )
                return 0

            lax.fori_loop(0, CHUNK // 16, body, 0)
            pltpu.sync_copy(obuf, out_hbm.at[pl.ds(r0, CHUNK)])

            @pl.when(c2 < CH_PW // 2 - 1)
            def _():
                pltpu.async_copy(tab_hbm.at[idx_v.at[c + 2]], buf, sem)

        return carry

    lax.fori_loop(0, CH_PW // 2, pair, 0)


def kernel(x, embed_weight, pe):
    x3 = x.reshape(NW, CH_PW, CHUNK).astype(jnp.int32)
    idx = x3 >> 1
    off = (x3 & 1) * DIM
    wt2 = embed_weight.reshape(500000, 2 * DIM)
    pe1 = pe[0, :SEQ].astype(jnp.float32)
    pe2 = jnp.concatenate([pe1, pe1], axis=0)
    out = _emb_kernel(idx, off, pe2, wt2)
    return out.reshape(BATCH, SEQ, DIM)
